# baseline (speedup 1.0000x reference)
import functools

import jax
import jax.numpy as jnp
import numpy as np
from jax import lax
from jax.experimental import pallas as pl
from jax.experimental.pallas import tpu as pltpu
from jax.experimental.pallas import tpu_sc as plsc

K = 16
SPOTS = 4
SPOT_K = 16
BLOCKS = 2
SIGMA_C = 3.0
SEED_NUM = 64
SEED_THR = 0.5
HEADS = 4
N = 1024
C = 256
DH = C // HEADS
NEG = -1e30
BIGI = 2 ** 30
F32 = jnp.float32


def _f32(shape):
    return jax.ShapeDtypeStruct(shape, jnp.float32)


def _i32(shape):
    return jax.ShapeDtypeStruct(shape, jnp.int32)


def _pdist_topk_body(pts_ref, d_ref, idx_ref):
    pts = pts_ref[...]
    cx, cy, cz = pts[:, 0:1], pts[:, 1:2], pts[:, 2:3]
    sq_col = cx * cx + cy * cy + cz * cz
    eye3 = jnp.eye(3, dtype=F32)
    rows = jax.lax.dot_general(eye3, pts, (((1,), (1,)), ((), ())),
                               precision=jax.lax.Precision.HIGHEST,
                               preferred_element_type=F32)
    rx, ry, rz = rows[0:1, :], rows[1:2, :], rows[2:3, :]
    sq_row = rx * rx + ry * ry + rz * rz
    dotm = jax.lax.dot_general(pts, pts, (((1,), (1,)), ((), ())),
                               preferred_element_type=F32)
    d = sq_col + sq_row - 2.0 * dotm
    d_ref[...] = d
    ji = jax.lax.broadcasted_iota(jnp.int32, (N, N), 1)
    jk = jax.lax.broadcasted_iota(jnp.int32, (N, 32), 1)
    neg = -d

    def body(j, carry):
        neg, idxv = carry
        m = jnp.max(neg, axis=1, keepdims=True)
        cand = jnp.where(neg == m, ji, BIGI)
        mink = jnp.min(cand, axis=1, keepdims=True)
        idxv = jnp.where(jk == j, mink, idxv)
        neg = jnp.where(ji == mink, NEG, neg)
        return neg, idxv

    _, idxv = jax.lax.fori_loop(0, K + 1, body, (neg, jnp.zeros((N, 32), jnp.int32)))
    idx_ref[...] = idxv


def _pdist_topk(pts):
    return pl.pallas_call(
        _pdist_topk_body,
        out_shape=(_f32((N, N)), _i32((N, 32))),
    )(pts)


def _linproj_body(x_ref, w_ref, b_ref, o_ref):
    o_ref[...] = jnp.dot(x_ref[...], w_ref[...],
                         preferred_element_type=F32) + b_ref[...]


def _linproj(x, w, b):
    n, fi = x.shape
    fo = w.shape[1]
    return pl.pallas_call(
        _linproj_body,
        out_shape=_f32((n, fo)),
    )(x, w, b.reshape(1, fo))


def _matching_body(a_ref, b_ref, ms_ref, cr_ref, mir_ref, cs_ref, mis_ref):
    s = jax.lax.dot_general(a_ref[...], b_ref[...], (((1,), (1,)), ((), ())),
                            preferred_element_type=F32)
    rmax = jnp.max(s, axis=1, keepdims=True)
    er = jnp.exp(s - rmax)
    p = er / jnp.sum(er, axis=1, keepdims=True)
    cmax = jnp.max(s, axis=0, keepdims=True)
    ec = jnp.exp(s - cmax)
    q = ec / jnp.sum(ec, axis=0, keepdims=True)
    ms = p * q
    ms_ref[...] = ms
    ji = jax.lax.broadcasted_iota(jnp.int32, (N, N), 1)
    jc = jax.lax.broadcasted_iota(jnp.int32, (N, N), 0)
    cr = jnp.max(ms, axis=1, keepdims=True)
    cr_ref[...] = cr
    mir_ref[...] = jnp.min(jnp.where(ms == cr, ji, BIGI), axis=1, keepdims=True)
    cs = jnp.max(ms, axis=0, keepdims=True)
    cs_ref[...] = cs
    mis_ref[...] = jnp.min(jnp.where(ms == cs, jc, BIGI), axis=0, keepdims=True)


def _matching(a, b):
    return pl.pallas_call(
        _matching_body,
        out_shape=(_f32((N, N)), _f32((N, 1)), _i32((N, 1)),
                   _f32((1, N)), _i32((1, N))),
    )(a, b)


def _compat_body(down_ref, doth_ref, mi_ref, comp_ref):
    ji = jax.lax.broadcasted_iota(jnp.int32, (N, N), 1)
    p = (ji == mi_ref[...]).astype(F32)
    hi = jax.lax.Precision.HIGHEST
    g1 = jnp.dot(p, doth_ref[...], precision=hi, preferred_element_type=F32)
    dg = jax.lax.dot_general(g1, p, (((1,), (1,)), ((), ())), precision=hi,
                             preferred_element_type=F32)
    c = jnp.maximum(1.0 - jnp.abs(down_ref[...] - dg) / SIGMA_C, 0.0)
    comp_ref[...] = jnp.mean(c, axis=1, keepdims=True)


def _compat_k(d_own, d_other, mi):
    return pl.pallas_call(
        _compat_body,
        out_shape=_f32((N, 1)),
    )(d_own, d_other, mi)


def _seeds_body(comp_ref, conf_ref, confr_ref, tok_ref):
    comp = comp_ref[...]
    conf_r = conf_ref[...] * comp
    confr_ref[...] = conf_r
    sel = jnp.where(comp < jnp.max(comp, axis=1, keepdims=True) * SEED_THR,
                    conf_r, 0.0)
    ji = jax.lax.broadcasted_iota(jnp.int32, (2, N), 1)
    jk = jax.lax.broadcasted_iota(jnp.int32, (2, SEED_NUM), 1)

    def body(j, carry):
        sel, tokv = carry
        m = jnp.max(sel, axis=1, keepdims=True)
        t = jnp.min(jnp.where(sel == m, ji, BIGI), axis=1, keepdims=True)
        tokv = jnp.where(jk == j, t, tokv)
        sel = jnp.where(ji == t, -1.0, sel)
        return sel, tokv

    _, tokv = jax.lax.fori_loop(0, SEED_NUM, body,
                                (sel, jnp.zeros((2, SEED_NUM), jnp.int32)))
    tok_ref[...] = tokv


def _seeds2(comp2, conf2):
    return pl.pallas_call(
        _seeds_body,
        out_shape=(_f32((2, N)), _i32((2, SEED_NUM))),
    )(comp2, conf2)


def _ln_in(x, s, b):
    m = jnp.mean(x, -1, keepdims=True)
    v = jnp.mean((x - m) ** 2, -1, keepdims=True)
    return (x - m) / jnp.sqrt(v + 1e-5) * s + b


def _attn_tail(x, out, p_refs):
    (wo, bo, n1s, n1b, f1w, f1b, f2w, f2b, n2s, n2b) = p_refs
    x1 = _ln_in(x + jnp.dot(out, wo[...], preferred_element_type=F32) + bo[...],
                n1s[...], n1b[...])
    h = jnp.maximum(jnp.dot(x1, f1w[...], preferred_element_type=F32) + f1b[...], 0.0)
    x2 = _ln_in(x1 + jnp.dot(h, f2w[...], preferred_element_type=F32) + f2b[...],
                n2s[...], n2b[...])
    return x2


def _caa_body(x_ref, tok_ref, wq, bq, wk, bk, wv, bv, wo, bo, n1s, n1b,
              f1w, f1b, f2w, f2b, n2s, n2b, o_ref, xs_ref):
    def gather(i, _):
        r = tok_ref[0, i]
        xs_ref[pl.ds(i, 1), :] = x_ref[pl.ds(r, 1), :]
        return 0

    jax.lax.fori_loop(0, SEED_NUM, gather, 0)
    x = x_ref[...]
    xs = xs_ref[...]
    q = jnp.dot(x, wq[...], preferred_element_type=F32) + bq[...]
    k = jnp.dot(xs, wk[...], preferred_element_type=F32) + bk[...]
    v = jnp.dot(xs, wv[...], preferred_element_type=F32) + bv[...]
    hi = jax.lax.Precision.HIGHEST
    outs = []
    for h in range(HEADS):
        sl = slice(h * DH, (h + 1) * DH)
        sh = jax.lax.dot_general(q[:, sl], k[:, sl], (((1,), (1,)), ((), ())),
                                 precision=hi,
                                 preferred_element_type=F32) / np.sqrt(DH)
        rm = jnp.max(sh, axis=1, keepdims=True)
        e = jnp.exp(sh - rm)
        ah = e / jnp.sum(e, axis=1, keepdims=True)
        outs.append(jnp.dot(ah, v[:, sl], precision=hi, preferred_element_type=F32))
    out = jnp.concatenate(outs, axis=1)
    o_ref[...] = _attn_tail(x, out, (wo, bo, n1s, n1b, f1w, f1b, f2w, f2b, n2s, n2b))


def _caa_layer(p, x, tok):
    return pl.pallas_call(
        _caa_body,
        out_shape=_f32((N, C)),
        in_specs=[pl.BlockSpec(memory_space=pltpu.VMEM),
                  pl.BlockSpec(memory_space=pltpu.SMEM)]
                 + [pl.BlockSpec(memory_space=pltpu.VMEM) for _ in range(16)],
        scratch_shapes=[pltpu.VMEM((SEED_NUM, C), F32)],
    )(x, tok,
      p["q"]["W"], p["q"]["b"].reshape(1, C),
      p["k"]["W"], p["k"]["b"].reshape(1, C),
      p["v"]["W"], p["v"]["b"].reshape(1, C),
      p["o"]["W"], p["o"]["b"].reshape(1, C),
      p["n1s"].reshape(1, C), p["n1b"].reshape(1, C),
      p["f1"]["W"], p["f1"]["b"].reshape(1, 2 * C),
      p["f2"]["W"], p["f2"]["b"].reshape(1, C),
      p["n2s"].reshape(1, C), p["n2b"].reshape(1, C))


def _sga_body(x_ref, mem_ref, cnt_ref, wq, bq, wk, bk, wv, bv, wo, bo,
              n1s, n1b, f1w, f1b, f2w, f2b, n2s, n2b, o_ref):
    x = x_ref[...]
    mem = mem_ref[...]
    cnt = cnt_ref[...]
    live = cnt > 0.0
    q = jnp.dot(x, wq[...], preferred_element_type=F32) + bq[...]
    k = jnp.dot(mem, wk[...], preferred_element_type=F32) + bk[...]
    v = jnp.dot(mem, wv[...], preferred_element_type=F32) + bv[...]
    hi = jax.lax.Precision.HIGHEST
    outs = []
    for h in range(HEADS):
        sl = slice(h * DH, (h + 1) * DH)
        sh = jax.lax.dot_general(q[:, sl], k[:, sl], (((1,), (1,)), ((), ())),
                                 precision=hi,
                                 preferred_element_type=F32) / np.sqrt(DH)
        m = jnp.max(jnp.where(live, sh, NEG), axis=1, keepdims=True)
        w = jnp.where(live, cnt * jnp.exp(sh - m), 0.0)
        ah = w / jnp.sum(w, axis=1, keepdims=True)
        outs.append(jnp.dot(ah, v[:, sl], precision=hi, preferred_element_type=F32))
    out = jnp.concatenate(outs, axis=1)
    o_ref[...] = _attn_tail(x, out, (wo, bo, n1s, n1b, f1w, f1b, f2w, f2b, n2s, n2b))


def _sga_layer(p, x, mem, cnt):
    return pl.pallas_call(
        _sga_body,
        out_shape=_f32((N, C)),
    )(x, mem, cnt,
      p["q"]["W"], p["q"]["b"].reshape(1, C),
      p["k"]["W"], p["k"]["b"].reshape(1, C),
      p["v"]["W"], p["v"]["b"].reshape(1, C),
      p["o"]["W"], p["o"]["b"].reshape(1, C),
      p["n1s"].reshape(1, C), p["n1b"].reshape(1, C),
      p["f1"]["W"], p["f1"]["b"].reshape(1, 2 * C),
      p["f2"]["W"], p["f2"]["b"].reshape(1, C),
      p["n2s"].reshape(1, C), p["n2b"].reshape(1, C))


_NW = 32
_RW = N // _NW


def _spots_sc_body(knn_hbm, cross_hbm, conf_hbm, mi_hbm, out_hbm,
                   knn_v, cross_v, conf_v, mi_v, out_v, crow, mrow):
    wid = lax.axis_index("s") * 2 + lax.axis_index("c")
    base = wid * _RW
    pltpu.sync_copy(knn_hbm.at[pl.ds(base * 32, _RW * 32)], knn_v)
    pltpu.sync_copy(cross_hbm.at[:], cross_v)
    pltpu.sync_copy(conf_hbm.at[:], conf_v)
    pltpu.sync_copy(mi_hbm.at[:], mi_v)
    iota = lax.iota(jnp.int32, 16)
    negv = jnp.full((16,), NEG, jnp.float32)
    mask0 = iota == 0

    def row(r, carry):
        off = r * 32
        idx_a = plsc.load_gather(knn_v, [off + iota])
        idx17 = plsc.load_gather(knn_v, [jnp.full((16,), 16, jnp.int32) + off])
        c_a = plsc.load_gather(conf_v, [idx_a])
        c17 = plsc.load_gather(conf_v, [idx17])
        m_a = plsc.load_gather(mi_v, [idx_a])
        m17 = plsc.load_gather(mi_v, [idx17])
        plsc.store_scatter(crow, [iota], c_a)
        plsc.store_scatter(crow, [iota + 16], jnp.where(mask0, c17, negv))
        plsc.store_scatter(mrow, [iota], m_a)
        plsc.store_scatter(mrow, [iota + 16], m17)

        def pick(k, carry2):
            ca = plsc.load_gather(crow, [iota])
            cb = plsc.load_gather(crow, [iota + 16])
            m = jnp.max(jnp.maximum(ca, cb))
            mask_a = ca == m
            has_a = plsc.all_reduce_population_count(mask_a) > 0
            pos_a = plsc.all_reduce_ffs(mask_a)
            pos_b = plsc.all_reduce_ffs(cb == m) + 16
            pos = jnp.where(has_a, pos_a, pos_b)
            center = plsc.load_gather(mrow, [pos])
            spot16 = plsc.load_gather(cross_v, [center * 32 + iota])
            plsc.store_scatter(out_v, [r * 64 + k * 16 + iota], spot16)
            plsc.store_scatter(crow, [pos], negv, mask=mask0)
            return carry2

        lax.fori_loop(0, SPOTS, pick, 0, unroll=True)
        return carry

    lax.fori_loop(0, _RW, row, 0)
    pltpu.sync_copy(out_v, out_hbm.at[pl.ds(base * 64, _RW * 64)])


def _compat_sc_body(down_hbm, doth_hbm, mi_hbm, comp_hbm,
                    mi_v, mi32_v, g_v, down_v, comp_v, sem):
    wid = lax.axis_index("s") * 2 + lax.axis_index("c")
    base = wid * _RW
    pltpu.sync_copy(mi_hbm.at[:], mi_v)
    pltpu.sync_copy(mi_hbm.at[pl.ds(base, _RW)], mi32_v)
    pltpu.async_copy(doth_hbm.at[mi32_v], g_v, sem).wait()
    pltpu.sync_copy(down_hbm.at[pl.ds(base, _RW)], down_v)
    iota = lax.iota(jnp.int32, 16)
    mask0 = iota == 0

    def row(r, c):
        rs = jnp.zeros((16,), jnp.int32) + r

        def col(j, acc):
            mc = plsc.load_gather(mi_v, [j * 16 + iota])
            g = plsc.load_gather(g_v, [rs, mc])
            dn = plsc.load_gather(down_v, [rs, j * 16 + iota])
            return acc + jnp.maximum(1.0 - jnp.abs(dn - g) / SIGMA_C, 0.0)

        acc = lax.fori_loop(0, N // 16, col, jnp.zeros((16,), jnp.float32))
        total = jnp.sum(acc) * (1.0 / N)
        plsc.store_scatter(comp_v, [rs], jnp.zeros((16,), jnp.float32) + total,
                           mask=mask0)
        return c

    lax.fori_loop(0, _RW, row, 0)
    pltpu.sync_copy(comp_v, comp_hbm.at[pl.ds(base, _RW)])


def _compat_sc(d_own, d_other, mi):
    mesh = plsc.VectorSubcoreMesh(core_axis_name="c", subcore_axis_name="s")
    kfn = functools.partial(
        pl.kernel, mesh=mesh,
        compiler_params=pltpu.CompilerParams(needs_layout_passes=False),
        out_type=jax.ShapeDtypeStruct((N,), jnp.float32),
        scratch_types=[
            pltpu.VMEM((N,), jnp.int32),
            pltpu.VMEM((_RW,), jnp.int32),
            pltpu.VMEM((_RW, N), jnp.float32),
            pltpu.VMEM((_RW, N), jnp.float32),
            pltpu.VMEM((_RW,), jnp.float32),
            pltpu.SemaphoreType.DMA,
        ],
    )(_compat_sc_body)
    return kfn(d_own, d_other, mi)


def _counts_sc_body(idx_hbm, out_hbm, idx_v, cnt_v):
    wid = lax.axis_index("s") * 2 + lax.axis_index("c")
    base = wid * _RW
    pltpu.sync_copy(idx_hbm.at[pl.ds(base * 64, _RW * 64)], idx_v)
    iota = lax.iota(jnp.int32, 16)
    zero = jnp.zeros((16,), jnp.float32)
    one = jnp.ones((16,), jnp.float32)

    def z(i, c):
        plsc.store_scatter(cnt_v, [i * 16 + iota], zero)
        return c

    lax.fori_loop(0, _RW * 64, z, 0)

    def row(r, c):
        def chunk(k, c2):
            t = plsc.load_gather(idx_v, [r * 64 + k * 16 + iota])
            plsc.addupdate_scatter(cnt_v, [r * 1024 + t], one)
            return c2

        lax.fori_loop(0, SPOTS, chunk, 0)
        return c

    lax.fori_loop(0, _RW, row, 0)
    pltpu.sync_copy(cnt_v, out_hbm.at[pl.ds(base * 1024, _RW * 1024)])


def _counts_sc(spot_flat):
    mesh = plsc.VectorSubcoreMesh(core_axis_name="c", subcore_axis_name="s")
    kfn = functools.partial(
        pl.kernel, mesh=mesh,
        compiler_params=pltpu.CompilerParams(needs_layout_passes=False),
        out_type=jax.ShapeDtypeStruct((N * N,), jnp.float32),
        scratch_types=[
            pltpu.VMEM((_RW * 64,), jnp.int32),
            pltpu.VMEM((_RW * N,), jnp.float32),
        ],
    )(_counts_sc_body)
    return kfn(spot_flat).reshape(N, N)


def _select_spots_sc(knn_flat, cross_flat, conf, mi):
    mesh = plsc.VectorSubcoreMesh(core_axis_name="c", subcore_axis_name="s")
    kfn = functools.partial(
        pl.kernel, mesh=mesh,
        compiler_params=pltpu.CompilerParams(needs_layout_passes=False),
        out_type=jax.ShapeDtypeStruct((N * 64,), jnp.int32),
        scratch_types=[
            pltpu.VMEM((_RW * 32,), jnp.int32),
            pltpu.VMEM((N * 32,), jnp.int32),
            pltpu.VMEM((N,), jnp.float32),
            pltpu.VMEM((N,), jnp.int32),
            pltpu.VMEM((_RW * 64,), jnp.int32),
            pltpu.VMEM((32,), jnp.float32),
            pltpu.VMEM((32,), jnp.int32),
        ],
    )(_spots_sc_body)
    return kfn(knn_flat, cross_flat, conf, mi).reshape(N, SPOTS * SPOT_K)


def kernel(ref_points, src_points, ref_feats, src_feats, params):
    ref_d, ref_idx = _pdist_topk(ref_points[0])
    src_d, src_idx = _pdist_topk(src_points[0])
    ref_f = _linproj(ref_feats[0], params["in_proj"]["W"], params["in_proj"]["b"])
    src_f = _linproj(src_feats[0], params["in_proj"]["W"], params["in_proj"]["b"])

    corr, ref_comps, src_comps = [], [], []
    new_ref = new_src = None
    for i in range(BLOCKS):
        last = (i == BLOCKS - 1)
        ms, conf_r_col, mi_r_col, conf_s_row, mi_s_row = _matching(ref_f, src_f)
        corr.append(ms)

        comp_r = _compat_sc(ref_d, src_d, mi_r_col[:, 0])
        comp_s = _compat_sc(src_d, ref_d, mi_s_row[0])
        comp2 = jnp.stack([comp_r, comp_s])
        conf2 = jnp.concatenate([conf_r_col.reshape(1, N), conf_s_row], axis=0)
        confr2, tok2 = _seeds2(comp2, conf2)
        conf_r_full, conf_s_full = confr2[0:1], confr2[1:2]
        ref_tok, src_tok = tok2[0:1], tok2[1:2]
        ref_comps.append(comp_r.reshape(1, N))
        src_comps.append(comp_s.reshape(1, N))

        if last:
            ref_spot = _select_spots_sc(ref_idx.reshape(-1), src_idx.reshape(-1),
                                        conf_r_full[0], mi_r_col[:, 0])
            src_spot = _select_spots_sc(src_idx.reshape(-1), ref_idx.reshape(-1),
                                        conf_s_full[0], mi_s_row[0])

        ref_f = _caa_layer(params["caa"][i], ref_f, ref_tok)
        src_f = _caa_layer(params["caa"][i], src_f, src_tok)
        if last:
            new_ref = _sga_layer(params["sga"][i], ref_f, src_f,
                                 _counts_sc(ref_spot.reshape(-1)))
            new_src = _sga_layer(params["sga"][i], src_f, ref_f,
                                 _counts_sc(src_spot.reshape(-1)))

    return (new_ref[None], new_src[None],
            jnp.stack(corr, -1)[None],
            jnp.stack([r[0] for r in ref_comps], -1)[None],
            jnp.stack([s[0] for s in src_comps], -1)[None])


# speedup vs baseline: 1.0437x; 1.0437x over previous
import functools

import jax
import jax.numpy as jnp
import numpy as np
from jax import lax
from jax.experimental import pallas as pl
from jax.experimental.pallas import tpu as pltpu
from jax.experimental.pallas import tpu_sc as plsc

K = 16
SPOTS = 4
SPOT_K = 16
BLOCKS = 2
SIGMA_C = 3.0
SEED_NUM = 64
SEED_THR = 0.5
HEADS = 4
N = 1024
C = 256
DH = C // HEADS
NEG = -1e30
BIGI = 2 ** 30
F32 = jnp.float32


def _f32(shape):
    return jax.ShapeDtypeStruct(shape, jnp.float32)


def _i32(shape):
    return jax.ShapeDtypeStruct(shape, jnp.int32)


def _pdist_topk_body(pts_ref, d_ref, idx_ref):
    pts = pts_ref[...]
    cx, cy, cz = pts[:, 0:1], pts[:, 1:2], pts[:, 2:3]
    sq_col = cx * cx + cy * cy + cz * cz
    eye3 = jnp.eye(3, dtype=F32)
    rows = jax.lax.dot_general(eye3, pts, (((1,), (1,)), ((), ())),
                               precision=jax.lax.Precision.HIGHEST,
                               preferred_element_type=F32)
    rx, ry, rz = rows[0:1, :], rows[1:2, :], rows[2:3, :]
    sq_row = rx * rx + ry * ry + rz * rz
    dotm = jax.lax.dot_general(pts, pts, (((1,), (1,)), ((), ())),
                               preferred_element_type=F32)
    d = sq_col + sq_row - 2.0 * dotm
    d_ref[...] = d
    ji = jax.lax.broadcasted_iota(jnp.int32, (N, N), 1)
    jk = jax.lax.broadcasted_iota(jnp.int32, (N, 32), 1)
    neg = -d

    def body(j, carry):
        neg, idxv = carry
        m = jnp.max(neg, axis=1, keepdims=True)
        cand = jnp.where(neg == m, ji, BIGI)
        mink = jnp.min(cand, axis=1, keepdims=True)
        idxv = jnp.where(jk == j, mink, idxv)
        neg = jnp.where(ji == mink, NEG, neg)
        return neg, idxv

    _, idxv = jax.lax.fori_loop(0, K + 1, body, (neg, jnp.zeros((N, 32), jnp.int32)))
    idx_ref[...] = idxv


def _pdist_topk(pts):
    return pl.pallas_call(
        _pdist_topk_body,
        out_shape=(_f32((N, N)), _i32((N, 32))),
    )(pts)


def _linproj_body(x_ref, w_ref, b_ref, o_ref):
    o_ref[...] = jnp.dot(x_ref[...], w_ref[...],
                         preferred_element_type=F32) + b_ref[...]


def _linproj(x, w, b):
    n, fi = x.shape
    fo = w.shape[1]
    return pl.pallas_call(
        _linproj_body,
        out_shape=_f32((n, fo)),
    )(x, w, b.reshape(1, fo))


def _matching_body(a_ref, b_ref, ms_ref, cr_ref, mir_ref, cs_ref, mis_ref):
    s = jax.lax.dot_general(a_ref[...], b_ref[...], (((1,), (1,)), ((), ())),
                            preferred_element_type=F32)
    rmax = jnp.max(s, axis=1, keepdims=True)
    er = jnp.exp(s - rmax)
    p = er / jnp.sum(er, axis=1, keepdims=True)
    cmax = jnp.max(s, axis=0, keepdims=True)
    ec = jnp.exp(s - cmax)
    q = ec / jnp.sum(ec, axis=0, keepdims=True)
    ms = p * q
    ms_ref[...] = ms
    ji = jax.lax.broadcasted_iota(jnp.int32, (N, N), 1)
    jc = jax.lax.broadcasted_iota(jnp.int32, (N, N), 0)
    cr = jnp.max(ms, axis=1, keepdims=True)
    cr_ref[...] = cr
    mir_ref[...] = jnp.min(jnp.where(ms == cr, ji, BIGI), axis=1, keepdims=True)
    cs = jnp.max(ms, axis=0, keepdims=True)
    cs_ref[...] = cs
    mis_ref[...] = jnp.min(jnp.where(ms == cs, jc, BIGI), axis=0, keepdims=True)


def _matching(a, b):
    return pl.pallas_call(
        _matching_body,
        out_shape=(_f32((N, N)), _f32((N, 1)), _i32((N, 1)),
                   _f32((1, N)), _i32((1, N))),
    )(a, b)


def _compat_body(down_ref, doth_ref, mi_ref, comp_ref):
    ji = jax.lax.broadcasted_iota(jnp.int32, (N, N), 1)
    p = (ji == mi_ref[...]).astype(F32)
    hi = jax.lax.Precision.HIGHEST
    g1 = jnp.dot(p, doth_ref[...], precision=hi, preferred_element_type=F32)
    dg = jax.lax.dot_general(g1, p, (((1,), (1,)), ((), ())), precision=hi,
                             preferred_element_type=F32)
    c = jnp.maximum(1.0 - jnp.abs(down_ref[...] - dg) / SIGMA_C, 0.0)
    comp_ref[...] = jnp.mean(c, axis=1, keepdims=True)


def _compat_k(d_own, d_other, mi):
    return pl.pallas_call(
        _compat_body,
        out_shape=_f32((N, 1)),
    )(d_own, d_other, mi)


def _seeds_body(comp_ref, conf_ref, confr_ref, tok_ref):
    comp = comp_ref[...]
    conf_r = conf_ref[...] * comp
    confr_ref[...] = conf_r
    sel = jnp.where(comp < jnp.max(comp, axis=1, keepdims=True) * SEED_THR,
                    conf_r, 0.0)
    ji = jax.lax.broadcasted_iota(jnp.int32, (1, N), 1)
    jk = jax.lax.broadcasted_iota(jnp.int32, (1, SEED_NUM), 1)

    def body(j, carry):
        sel, tokv = carry
        m = jnp.max(sel, axis=1, keepdims=True)
        t = jnp.min(jnp.where(sel == m, ji, BIGI), axis=1, keepdims=True)
        tokv = jnp.where(jk == j, t, tokv)
        sel = jnp.where(ji == t, -1.0, sel)
        return sel, tokv

    _, tokv = jax.lax.fori_loop(0, SEED_NUM, body,
                                (sel, jnp.zeros((1, SEED_NUM), jnp.int32)))
    tok_ref[...] = tokv


def _seeds(comp, conf):
    return pl.pallas_call(
        _seeds_body,
        out_shape=(_f32((1, N)), _i32((1, SEED_NUM))),
    )(comp, conf)


def _ln_in(x, s, b):
    m = jnp.mean(x, -1, keepdims=True)
    v = jnp.mean((x - m) ** 2, -1, keepdims=True)
    return (x - m) / jnp.sqrt(v + 1e-5) * s + b


def _attn_tail(x, out, p_refs):
    (wo, bo, n1s, n1b, f1w, f1b, f2w, f2b, n2s, n2b) = p_refs
    x1 = _ln_in(x + jnp.dot(out, wo[...], preferred_element_type=F32) + bo[...],
                n1s[...], n1b[...])
    h = jnp.maximum(jnp.dot(x1, f1w[...], preferred_element_type=F32) + f1b[...], 0.0)
    x2 = _ln_in(x1 + jnp.dot(h, f2w[...], preferred_element_type=F32) + f2b[...],
                n2s[...], n2b[...])
    return x2


def _caa_body(x_ref, tok_ref, wq, bq, wk, bk, wv, bv, wo, bo, n1s, n1b,
              f1w, f1b, f2w, f2b, n2s, n2b, o_ref, xs_ref):
    def gather(i, _):
        r = tok_ref[0, i]
        xs_ref[pl.ds(i, 1), :] = x_ref[pl.ds(r, 1), :]
        return 0

    jax.lax.fori_loop(0, SEED_NUM, gather, 0)
    x = x_ref[...]
    xs = xs_ref[...]
    q = jnp.dot(x, wq[...], preferred_element_type=F32) + bq[...]
    k = jnp.dot(xs, wk[...], preferred_element_type=F32) + bk[...]
    v = jnp.dot(xs, wv[...], preferred_element_type=F32) + bv[...]
    hi = jax.lax.Precision.HIGHEST
    outs = []
    for h in range(HEADS):
        sl = slice(h * DH, (h + 1) * DH)
        sh = jax.lax.dot_general(q[:, sl], k[:, sl], (((1,), (1,)), ((), ())),
                                 precision=hi,
                                 preferred_element_type=F32) / np.sqrt(DH)
        rm = jnp.max(sh, axis=1, keepdims=True)
        e = jnp.exp(sh - rm)
        ah = e / jnp.sum(e, axis=1, keepdims=True)
        outs.append(jnp.dot(ah, v[:, sl], precision=hi, preferred_element_type=F32))
    out = jnp.concatenate(outs, axis=1)
    o_ref[...] = _attn_tail(x, out, (wo, bo, n1s, n1b, f1w, f1b, f2w, f2b, n2s, n2b))


def _caa_layer(p, x, tok):
    return pl.pallas_call(
        _caa_body,
        out_shape=_f32((N, C)),
        in_specs=[pl.BlockSpec(memory_space=pltpu.VMEM),
                  pl.BlockSpec(memory_space=pltpu.SMEM)]
                 + [pl.BlockSpec(memory_space=pltpu.VMEM) for _ in range(16)],
        scratch_shapes=[pltpu.VMEM((SEED_NUM, C), F32)],
    )(x, tok,
      p["q"]["W"], p["q"]["b"].reshape(1, C),
      p["k"]["W"], p["k"]["b"].reshape(1, C),
      p["v"]["W"], p["v"]["b"].reshape(1, C),
      p["o"]["W"], p["o"]["b"].reshape(1, C),
      p["n1s"].reshape(1, C), p["n1b"].reshape(1, C),
      p["f1"]["W"], p["f1"]["b"].reshape(1, 2 * C),
      p["f2"]["W"], p["f2"]["b"].reshape(1, C),
      p["n2s"].reshape(1, C), p["n2b"].reshape(1, C))


def _sga_body(x_ref, mem_ref, cnt_ref, wq, bq, wk, bk, wv, bv, wo, bo,
              n1s, n1b, f1w, f1b, f2w, f2b, n2s, n2b, o_ref):
    x = x_ref[...]
    mem = mem_ref[...]
    cnt = cnt_ref[...]
    live = cnt > 0.0
    q = jnp.dot(x, wq[...], preferred_element_type=F32) + bq[...]
    k = jnp.dot(mem, wk[...], preferred_element_type=F32) + bk[...]
    v = jnp.dot(mem, wv[...], preferred_element_type=F32) + bv[...]
    hi = jax.lax.Precision.HIGHEST
    outs = []
    for h in range(HEADS):
        sl = slice(h * DH, (h + 1) * DH)
        sh = jax.lax.dot_general(q[:, sl], k[:, sl], (((1,), (1,)), ((), ())),
                                 precision=hi,
                                 preferred_element_type=F32) / np.sqrt(DH)
        m = jnp.max(jnp.where(live, sh, NEG), axis=1, keepdims=True)
        w = jnp.where(live, cnt * jnp.exp(sh - m), 0.0)
        ah = w / jnp.sum(w, axis=1, keepdims=True)
        outs.append(jnp.dot(ah, v[:, sl], precision=hi, preferred_element_type=F32))
    out = jnp.concatenate(outs, axis=1)
    o_ref[...] = _attn_tail(x, out, (wo, bo, n1s, n1b, f1w, f1b, f2w, f2b, n2s, n2b))


def _sga_layer(p, x, mem, cnt):
    return pl.pallas_call(
        _sga_body,
        out_shape=_f32((N, C)),
    )(x, mem, cnt,
      p["q"]["W"], p["q"]["b"].reshape(1, C),
      p["k"]["W"], p["k"]["b"].reshape(1, C),
      p["v"]["W"], p["v"]["b"].reshape(1, C),
      p["o"]["W"], p["o"]["b"].reshape(1, C),
      p["n1s"].reshape(1, C), p["n1b"].reshape(1, C),
      p["f1"]["W"], p["f1"]["b"].reshape(1, 2 * C),
      p["f2"]["W"], p["f2"]["b"].reshape(1, C),
      p["n2s"].reshape(1, C), p["n2b"].reshape(1, C))


_NW = 32
_RW = N // _NW


def _spots_sc_body(knn_hbm, cross_hbm, conf_hbm, mi_hbm, out_hbm,
                   knn_v, cross_v, conf_v, mi_v, out_v, crow, mrow):
    wid = lax.axis_index("s") * 2 + lax.axis_index("c")
    base = wid * _RW
    pltpu.sync_copy(knn_hbm.at[pl.ds(base * 32, _RW * 32)], knn_v)
    pltpu.sync_copy(cross_hbm.at[:], cross_v)
    pltpu.sync_copy(conf_hbm.at[:], conf_v)
    pltpu.sync_copy(mi_hbm.at[:], mi_v)
    iota = lax.iota(jnp.int32, 16)
    negv = jnp.full((16,), NEG, jnp.float32)
    mask0 = iota == 0

    def row(r, carry):
        off = r * 32
        idx_a = plsc.load_gather(knn_v, [off + iota])
        idx17 = plsc.load_gather(knn_v, [jnp.full((16,), 16, jnp.int32) + off])
        c_a = plsc.load_gather(conf_v, [idx_a])
        c17 = plsc.load_gather(conf_v, [idx17])
        m_a = plsc.load_gather(mi_v, [idx_a])
        m17 = plsc.load_gather(mi_v, [idx17])
        plsc.store_scatter(crow, [iota], c_a)
        plsc.store_scatter(crow, [iota + 16], jnp.where(mask0, c17, negv))
        plsc.store_scatter(mrow, [iota], m_a)
        plsc.store_scatter(mrow, [iota + 16], m17)

        def pick(k, carry2):
            ca = plsc.load_gather(crow, [iota])
            cb = plsc.load_gather(crow, [iota + 16])
            m = jnp.max(jnp.maximum(ca, cb))
            mask_a = ca == m
            has_a = plsc.all_reduce_population_count(mask_a) > 0
            pos_a = plsc.all_reduce_ffs(mask_a)
            pos_b = plsc.all_reduce_ffs(cb == m) + 16
            pos = jnp.where(has_a, pos_a, pos_b)
            center = plsc.load_gather(mrow, [pos])
            spot16 = plsc.load_gather(cross_v, [center * 32 + iota])
            plsc.store_scatter(out_v, [r * 64 + k * 16 + iota], spot16)
            plsc.store_scatter(crow, [pos], negv, mask=mask0)
            return carry2

        lax.fori_loop(0, SPOTS, pick, 0, unroll=True)
        return carry

    lax.fori_loop(0, _RW, row, 0)
    pltpu.sync_copy(out_v, out_hbm.at[pl.ds(base * 64, _RW * 64)])


def _compat_sc_body(down_hbm, doth_hbm, mi_hbm, comp_hbm,
                    mi_v, mi32_v, g_v, down_v, comp_v, sem):
    wid = lax.axis_index("s") * 2 + lax.axis_index("c")
    base = wid * _RW
    pltpu.sync_copy(mi_hbm.at[:], mi_v)
    pltpu.sync_copy(mi_hbm.at[pl.ds(base, _RW)], mi32_v)
    pltpu.async_copy(doth_hbm.at[mi32_v], g_v, sem).wait()
    pltpu.sync_copy(down_hbm.at[pl.ds(base, _RW)], down_v)
    iota = lax.iota(jnp.int32, 16)
    mask0 = iota == 0

    def row(r, c):
        rs = jnp.zeros((16,), jnp.int32) + r

        def col(j, acc):
            mc = plsc.load_gather(mi_v, [j * 16 + iota])
            g = plsc.load_gather(g_v, [rs, mc])
            dn = plsc.load_gather(down_v, [rs, j * 16 + iota])
            return acc + jnp.maximum(1.0 - jnp.abs(dn - g) / SIGMA_C, 0.0)

        acc = lax.fori_loop(0, N // 16, col, jnp.zeros((16,), jnp.float32))
        total = jnp.sum(acc) * (1.0 / N)
        plsc.store_scatter(comp_v, [rs], jnp.zeros((16,), jnp.float32) + total,
                           mask=mask0)
        return c

    lax.fori_loop(0, _RW, row, 0)
    pltpu.sync_copy(comp_v, comp_hbm.at[pl.ds(base, _RW)])


def _compat_sc(d_own, d_other, mi):
    mesh = plsc.VectorSubcoreMesh(core_axis_name="c", subcore_axis_name="s")
    kfn = functools.partial(
        pl.kernel, mesh=mesh,
        compiler_params=pltpu.CompilerParams(needs_layout_passes=False),
        out_type=jax.ShapeDtypeStruct((N,), jnp.float32),
        scratch_types=[
            pltpu.VMEM((N,), jnp.int32),
            pltpu.VMEM((_RW,), jnp.int32),
            pltpu.VMEM((_RW, N), jnp.float32),
            pltpu.VMEM((_RW, N), jnp.float32),
            pltpu.VMEM((_RW,), jnp.float32),
            pltpu.SemaphoreType.DMA,
        ],
    )(_compat_sc_body)
    return kfn(d_own, d_other, mi)


def _counts_sc_body(idx_hbm, out_hbm, idx_v, cnt_v):
    wid = lax.axis_index("s") * 2 + lax.axis_index("c")
    base = wid * _RW
    pltpu.sync_copy(idx_hbm.at[pl.ds(base * 64, _RW * 64)], idx_v)
    iota = lax.iota(jnp.int32, 16)
    zero = jnp.zeros((16,), jnp.float32)
    one = jnp.ones((16,), jnp.float32)

    def z(i, c):
        plsc.store_scatter(cnt_v, [i * 16 + iota], zero)
        return c

    lax.fori_loop(0, _RW * 64, z, 0)

    def row(r, c):
        def chunk(k, c2):
            t = plsc.load_gather(idx_v, [r * 64 + k * 16 + iota])
            plsc.addupdate_scatter(cnt_v, [r * 1024 + t], one)
            return c2

        lax.fori_loop(0, SPOTS, chunk, 0)
        return c

    lax.fori_loop(0, _RW, row, 0)
    pltpu.sync_copy(cnt_v, out_hbm.at[pl.ds(base * 1024, _RW * 1024)])


def _counts_sc(spot_flat):
    mesh = plsc.VectorSubcoreMesh(core_axis_name="c", subcore_axis_name="s")
    kfn = functools.partial(
        pl.kernel, mesh=mesh,
        compiler_params=pltpu.CompilerParams(needs_layout_passes=False),
        out_type=jax.ShapeDtypeStruct((N * N,), jnp.float32),
        scratch_types=[
            pltpu.VMEM((_RW * 64,), jnp.int32),
            pltpu.VMEM((_RW * N,), jnp.float32),
        ],
    )(_counts_sc_body)
    return kfn(spot_flat).reshape(N, N)


def _select_spots_sc(knn_flat, cross_flat, conf, mi):
    mesh = plsc.VectorSubcoreMesh(core_axis_name="c", subcore_axis_name="s")
    kfn = functools.partial(
        pl.kernel, mesh=mesh,
        compiler_params=pltpu.CompilerParams(needs_layout_passes=False),
        out_type=jax.ShapeDtypeStruct((N * 64,), jnp.int32),
        scratch_types=[
            pltpu.VMEM((_RW * 32,), jnp.int32),
            pltpu.VMEM((N * 32,), jnp.int32),
            pltpu.VMEM((N,), jnp.float32),
            pltpu.VMEM((N,), jnp.int32),
            pltpu.VMEM((_RW * 64,), jnp.int32),
            pltpu.VMEM((32,), jnp.float32),
            pltpu.VMEM((32,), jnp.int32),
        ],
    )(_spots_sc_body)
    return kfn(knn_flat, cross_flat, conf, mi).reshape(N, SPOTS * SPOT_K)


def kernel(ref_points, src_points, ref_feats, src_feats, params):
    ref_d, ref_idx = _pdist_topk(ref_points[0])
    src_d, src_idx = _pdist_topk(src_points[0])
    ref_f = _linproj(ref_feats[0], params["in_proj"]["W"], params["in_proj"]["b"])
    src_f = _linproj(src_feats[0], params["in_proj"]["W"], params["in_proj"]["b"])

    corr, ref_comps, src_comps = [], [], []
    new_ref = new_src = None
    for i in range(BLOCKS):
        last = (i == BLOCKS - 1)
        ms, conf_r_col, mi_r_col, conf_s_row, mi_s_row = _matching(ref_f, src_f)
        corr.append(ms)

        comp_r = _compat_sc(ref_d, src_d, mi_r_col[:, 0])
        conf_r_full, ref_tok = _seeds(comp_r.reshape(1, N), conf_r_col.reshape(1, N))
        ref_comps.append(comp_r.reshape(1, N))

        comp_s = _compat_sc(src_d, ref_d, mi_s_row[0])
        conf_s_full, src_tok = _seeds(comp_s.reshape(1, N), conf_s_row)
        src_comps.append(comp_s.reshape(1, N))

        if last:
            ref_spot = _select_spots_sc(ref_idx.reshape(-1), src_idx.reshape(-1),
                                        conf_r_full[0], mi_r_col[:, 0])
            src_spot = _select_spots_sc(src_idx.reshape(-1), ref_idx.reshape(-1),
                                        conf_s_full[0], mi_s_row[0])

        ref_f = _caa_layer(params["caa"][i], ref_f, ref_tok)
        src_f = _caa_layer(params["caa"][i], src_f, src_tok)
        if last:
            new_ref = _sga_layer(params["sga"][i], ref_f, src_f,
                                 _counts_sc(ref_spot.reshape(-1)))
            new_src = _sga_layer(params["sga"][i], src_f, ref_f,
                                 _counts_sc(src_spot.reshape(-1)))

    return (new_ref[None], new_src[None],
            jnp.stack(corr, -1)[None],
            jnp.stack([r[0] for r in ref_comps], -1)[None],
            jnp.stack([s[0] for s in src_comps], -1)[None])
